# baseline (speedup 1.0000x reference)
import functools
import math

import jax
import jax.numpy as jnp
from jax import lax
from jax.experimental import pallas as pl
from jax.experimental.pallas import tpu as pltpu
from jax.experimental.pallas import tpu_sc as plsc

VOCAB = 100000
EMBED = 128
BATCH = 4096
SEQ = 50

NC, NS = 2, 16
NW = NC * NS
NSPLIT = 4
NBATCH = BATCH // NSPLIT
B_PER_W = NBATCH // NW
NBUF = 8
LOOKAHEAD = 5

SCALE = math.sqrt(float(EMBED))

_mesh = plsc.VectorSubcoreMesh(core_axis_name="c", subcore_axis_name="s")


@functools.partial(
    pl.kernel,
    mesh=_mesh,
    out_type=jax.ShapeDtypeStruct((NBATCH, SEQ, EMBED), jnp.float32),
    scratch_types=[
        pltpu.VMEM((B_PER_W, SEQ), jnp.int32),
        pltpu.VMEM((NBUF, SEQ, EMBED), jnp.float32),
        pltpu.SemaphoreType.DMA,
        pltpu.SemaphoreType.DMA,
        pltpu.SemaphoreType.DMA,
        pltpu.SemaphoreType.DMA,
        pltpu.SemaphoreType.DMA,
        pltpu.SemaphoreType.DMA,
        pltpu.SemaphoreType.DMA,
        pltpu.SemaphoreType.DMA,
        pltpu.SemaphoreType.DMA,
        pltpu.SemaphoreType.DMA,
        pltpu.SemaphoreType.DMA,
        pltpu.SemaphoreType.DMA,
        pltpu.SemaphoreType.DMA,
        pltpu.SemaphoreType.DMA,
        pltpu.SemaphoreType.DMA,
        pltpu.SemaphoreType.DMA,
    ],
)
def _embed_gather(table_hbm, x_hbm, out_hbm, idx_v, ring, *sems):
    wid = lax.axis_index("s") * NC + lax.axis_index("c")
    batch0 = wid * B_PER_W
    gsems = list(sems[:NBUF])
    ssems = list(sems[NBUF:])

    pltpu.sync_copy(x_hbm.at[pl.ds(batch0, B_PER_W)], idx_v)

    def gather_start(j, b):
        pltpu.async_copy(table_hbm.at[idx_v.at[j]], ring.at[b], gsems[b])

    def gather_wait(b):
        pltpu.make_async_copy(table_hbm.at[idx_v.at[0]], ring.at[b],
                              gsems[b]).wait()

    def scatter_start(j, b):
        pltpu.async_copy(ring.at[b], out_hbm.at[batch0 + j], ssems[b])

    def scatter_wait(b):
        pltpu.make_async_copy(ring.at[b], out_hbm.at[batch0], ssems[b]).wait()

    for j in range(LOOKAHEAD):
        gather_start(j, j)

    def visit(j, b):
        nj = j + LOOKAHEAD
        b2 = (b + LOOKAHEAD) % NBUF
        scatter_wait(b2)
        gather_start(nj, b2)
        gather_wait(b)
        scatter_start(j, b)

    for j in range(NBUF):
        b = j % NBUF
        if j < NBUF - LOOKAHEAD:
            gather_start(j + LOOKAHEAD, (b + LOOKAHEAD) % NBUF)
            gather_wait(b)
            scatter_start(j, b)
        else:
            visit(j, b)

    def group_body(g, carry):
        for b in range(NBUF):
            visit(g * NBUF + b, b)
        return carry

    lax.fori_loop(1, B_PER_W // NBUF - 1, group_body, 0)

    for j in range(B_PER_W - NBUF, B_PER_W):
        b = j % NBUF
        if j + LOOKAHEAD < B_PER_W:
            visit(j, b)
        else:
            gather_wait(b)
            scatter_start(j, b)

    for b in range(NBUF):
        scatter_wait(b)


def kernel(x, table):
    x32 = x.astype(jnp.int32)
    scaled_table = table * jnp.float32(SCALE)
    parts = [
        _embed_gather(scaled_table,
                      lax.slice_in_dim(x32, s * NBATCH, (s + 1) * NBATCH, axis=0))
        for s in range(NSPLIT)
    ]
    return jnp.concatenate(parts, axis=0)


# speedup vs baseline: 1.9231x; 1.9231x over previous
import functools
import math

import jax
import jax.numpy as jnp
from jax import lax
from jax.experimental import pallas as pl
from jax.experimental.pallas import tpu as pltpu
from jax.experimental.pallas import tpu_sc as plsc

VOCAB = 100000
EMBED = 128
BATCH = 4096
SEQ = 50

NC, NS = 2, 16
NW = NC * NS
NSPLIT = 1
NBATCH = BATCH // NSPLIT
B_PER_W = NBATCH // NW
NBUF = 8
LOOKAHEAD = 5

SCALE = math.sqrt(float(EMBED))

_mesh = plsc.VectorSubcoreMesh(core_axis_name="c", subcore_axis_name="s")


@functools.partial(
    pl.kernel,
    mesh=_mesh,
    out_type=jax.ShapeDtypeStruct((NBATCH, SEQ, EMBED), jnp.float32),
    scratch_types=[
        pltpu.VMEM((B_PER_W, SEQ), jnp.int32),
        pltpu.VMEM((NBUF, SEQ, EMBED), jnp.float32),
        pltpu.SemaphoreType.DMA,
        pltpu.SemaphoreType.DMA,
        pltpu.SemaphoreType.DMA,
        pltpu.SemaphoreType.DMA,
        pltpu.SemaphoreType.DMA,
        pltpu.SemaphoreType.DMA,
        pltpu.SemaphoreType.DMA,
        pltpu.SemaphoreType.DMA,
        pltpu.SemaphoreType.DMA,
        pltpu.SemaphoreType.DMA,
        pltpu.SemaphoreType.DMA,
        pltpu.SemaphoreType.DMA,
        pltpu.SemaphoreType.DMA,
        pltpu.SemaphoreType.DMA,
        pltpu.SemaphoreType.DMA,
        pltpu.SemaphoreType.DMA,
    ],
)
def _embed_gather(table_hbm, x_hbm, out_hbm, idx_v, ring, *sems):
    wid = lax.axis_index("s") * NC + lax.axis_index("c")
    batch0 = wid * B_PER_W
    gsems = list(sems[:NBUF])
    ssems = list(sems[NBUF:])

    pltpu.sync_copy(x_hbm.at[pl.ds(batch0, B_PER_W)], idx_v)

    def gather_start(j, b):
        pltpu.async_copy(table_hbm.at[idx_v.at[j]], ring.at[b], gsems[b])

    def gather_wait(b):
        pltpu.make_async_copy(table_hbm.at[idx_v.at[0]], ring.at[b],
                              gsems[b]).wait()

    def scatter_start(j, b):
        pltpu.async_copy(ring.at[b], out_hbm.at[batch0 + j], ssems[b])

    def scatter_wait(b):
        pltpu.make_async_copy(ring.at[b], out_hbm.at[batch0], ssems[b]).wait()

    def scale_slot(b):
        rb = ring.at[b]

        def row_body(r, c2):
            for k in range(EMBED // 16):
                sl = pl.ds(k * 16, 16)
                rb[r, sl] = rb[r, sl] * SCALE
            return c2

        lax.fori_loop(0, SEQ, row_body, 0, unroll=5)

    for j in range(LOOKAHEAD):
        gather_start(j, j)

    def visit(j, b):
        nj = j + LOOKAHEAD
        b2 = (b + LOOKAHEAD) % NBUF
        scatter_wait(b2)
        gather_start(nj, b2)
        gather_wait(b)
        scale_slot(b)
        scatter_start(j, b)

    for j in range(NBUF):
        b = j % NBUF
        if j < NBUF - LOOKAHEAD:
            gather_start(j + LOOKAHEAD, (b + LOOKAHEAD) % NBUF)
            gather_wait(b)
            scale_slot(b)
            scatter_start(j, b)
        else:
            visit(j, b)

    def group_body(g, carry):
        for b in range(NBUF):
            visit(g * NBUF + b, b)
        return carry

    lax.fori_loop(1, B_PER_W // NBUF - 1, group_body, 0)

    for j in range(B_PER_W - NBUF, B_PER_W):
        b = j % NBUF
        if j + LOOKAHEAD < B_PER_W:
            visit(j, b)
        else:
            gather_wait(b)
            scale_slot(b)
            scatter_start(j, b)

    for b in range(NBUF):
        scatter_wait(b)


def kernel(x, table):
    x32 = x.astype(jnp.int32)
    parts = [
        _embed_gather(table,
                      lax.slice_in_dim(x32, s * NBATCH, (s + 1) * NBATCH, axis=0))
        for s in range(NSPLIT)
    ]
    return jnp.concatenate(parts, axis=0)
